# baseline (speedup 1.0000x reference)
import functools

import jax
import jax.numpy as jnp
from jax import lax
from jax.experimental import pallas as pl
from jax.experimental.pallas import tpu as pltpu
from jax.experimental.pallas import tpu_sc as plsc

N = 10000
E = 320000
ND = 128
ED = 16
HD = 128
GW = HD

NC = 2
NS = 16
NW = NC * NS
NH = 2
EH = E // NH
EC = EH // NW
CH = 40
KCH = EC // CH
NPAD = 10240
NPT = NPAD // NS

RB = 2000
EB = 1600

_mesh = plsc.VectorSubcoreMesh(core_axis_name="c", subcore_axis_name="s")


def _node_pre_body(x_ref, nt_ref, w1, b1, w2, b2, gwx, gwt, out_ref):
    x = x_ref[...]
    h1 = jnp.maximum(jnp.dot(x, w1[...], preferred_element_type=jnp.float32)
                     + b1[...], 0.0)
    h_node = jnp.dot(h1, w2[...], preferred_element_type=jnp.float32) + b2[...]
    gp = (jnp.dot(x, gwx[...], preferred_element_type=jnp.float32)
          + nt_ref[...] * gwt[...])
    hu = jax.lax.bitcast_convert_type(h_node, jnp.uint32)
    gu = jax.lax.bitcast_convert_type(gp, jnp.uint32)
    rnd = lambda u: (u + jnp.uint32(0x7FFF) + ((u >> 16) & jnp.uint32(1))) >> 16
    packed = rnd(hu) | (rnd(gu) << 16)
    out_ref[...] = jax.lax.bitcast_convert_type(packed, jnp.float32)


def _edge_body(ea_ref, g_ref, ew1, eb1, ew2, eb2, gwe, gb1, gw2, gb2, mw, mb,
               out_ref):
    dot = functools.partial(jnp.dot, preferred_element_type=jnp.float32)
    ea8 = ea_ref[...]
    gu = jax.lax.bitcast_convert_type(g_ref[...], jnp.uint32)
    hn = jax.lax.bitcast_convert_type(gu << 16, jnp.float32)
    gp_node = jax.lax.bitcast_convert_type(gu & jnp.uint32(0xFFFF0000),
                                           jnp.float32)
    c1 = dot(ea8, ew1[...]).reshape(EB, HD)
    c2 = dot(ea8, gwe[...]).reshape(EB, HD)
    h1 = jnp.maximum(c1 + eb1[...], 0.0)
    he = dot(h1, ew2[...]) + eb2[...]
    gh = jnp.maximum(c2 + gp_node + gb1[...], 0.0)
    gate = dot(gh, gw2[...]) + gb2[...]
    m = dot(he * hn, mw[...]) + mb[...]
    out_ref[...] = m * jax.nn.sigmoid(gate)


def _final_body(x_ref, p0_ref, p1_ref, p2_ref, p3_ref, clw, clb, lng, lnb,
                ow, ob, out_ref):
    x = x_ref[...]
    o = (jnp.dot(x, clw[...], preferred_element_type=jnp.float32) + clb[...]
         + (p0_ref[0] + p1_ref[0]) + (p2_ref[0] + p3_ref[0]))
    mu = jnp.mean(o, axis=1, keepdims=True)
    var = jnp.mean((o - mu) * (o - mu), axis=1, keepdims=True)
    o = (o - mu) / jnp.sqrt(var + 1e-5) * lng[...] + lnb[...]
    o = jnp.maximum(o, 0.0)
    out_ref[...] = jnp.dot(o, ow[...], preferred_element_type=jnp.float32) + ob[...]


G = 5
GS = 5
CHS = 40
KCHS = EC // CHS


@functools.partial(
    pl.kernel,
    mesh=_mesh,
    out_type=jax.ShapeDtypeStruct((EH, GW), jnp.float32),
    scratch_types=[
        pltpu.VMEM((KCH, CH), jnp.int32),
        pltpu.VMEM((G, CH, GW), jnp.float32),
        pltpu.SemaphoreType.DMA,
    ],
)
def _sc_gather(table_hbm, idx_hbm, out_hbm, idx_v, rows_v, sem):
    c = lax.axis_index("c")
    s = lax.axis_index("s")
    t = c * NS + s
    pltpu.sync_copy(idx_hbm.at[t], idx_v)

    def group(g, carry):
        base = g * G
        handles = [
            pltpu.async_copy(table_hbm.at[idx_v.at[base + b]], rows_v.at[b],
                             sem)
            for b in range(G)
        ]
        for b in range(G):
            handles[b].wait()
            pltpu.sync_copy(rows_v.at[b],
                            out_hbm.at[pl.ds(t * EC + (base + b) * CH, CH)])
        return carry

    lax.fori_loop(0, KCH // G, group, 0)


@functools.partial(
    pl.kernel,
    mesh=_mesh,
    out_type=jax.ShapeDtypeStruct((NC * NPAD, HD), jnp.float32),
    scratch_types=[
        pltpu.VMEM((GS, CHS), jnp.int32),
        pltpu.VMEM((GS, CHS, HD), jnp.float32),
        pltpu.VMEM_SHARED((NPAD, HD), jnp.float32),
        pltpu.SemaphoreType.DMA,
        pltpu.SemaphoreType.DMA,
    ],
)
def _sc_scatter(msg_hbm, row_hbm, zero_hbm, out_hbm, idx_v, msg_v, acc_sh,
                sem, isem):
    c = lax.axis_index("c")
    s = lax.axis_index("s")
    t = c * NS + s
    pltpu.sync_copy(zero_hbm.at[pl.ds(s * NPT, NPT)],
                    acc_sh.at[pl.ds(s * NPT, NPT)])
    plsc.subcore_barrier()

    def group(g, carry):
        base = g * GS
        ih = pltpu.async_copy(row_hbm.at[t, g], idx_v, isem)
        handles = [
            pltpu.async_copy(
                msg_hbm.at[pl.ds(t * EC + (base + b) * CHS, CHS)],
                msg_v.at[b], sem)
            for b in range(GS)
        ]
        ih.wait()
        for b in range(GS):
            handles[b].wait()
            pltpu.sync_copy(msg_v.at[b], acc_sh.at[idx_v.at[b]],
                            add=True)
        return carry

    lax.fori_loop(0, KCHS // GS, group, 0)
    plsc.subcore_barrier()
    pltpu.sync_copy(acc_sh.at[pl.ds(s * NPT, NPT)],
                    out_hbm.at[pl.ds(c * NPAD + s * NPT, NPT)])


def kernel(x, edge_index, edge_attr, node_time,
           nn_W1, nn_b1, nn_W2, nn_b2,
           en_W1, en_b1, en_W2, en_b2,
           msg_W, msg_b,
           g_W1, g_b1, g_W2, g_b2,
           cl_W, cl_b, ln_g, ln_b, out_W, out_b):
    row = edge_index[0].reshape(NH, NW, KCHS // GS, GS, CHS)
    col = edge_index[1].reshape(NH, NW, KCH, CH)

    g_W1e = g_W1[:ED]
    g_W1x = g_W1[ED:ED + ND]
    g_W1t = g_W1[ED + ND:]

    r2 = lambda v: v.reshape(1, -1)

    table = pl.pallas_call(
        _node_pre_body,
        grid=(N // RB,),
        in_specs=[
            pl.BlockSpec((RB, ND), lambda i: (i, 0)),
            pl.BlockSpec((RB, 1), lambda i: (i, 0)),
            pl.BlockSpec((ND, HD), lambda i: (0, 0)),
            pl.BlockSpec((1, HD), lambda i: (0, 0)),
            pl.BlockSpec((HD, HD), lambda i: (0, 0)),
            pl.BlockSpec((1, HD), lambda i: (0, 0)),
            pl.BlockSpec((ND, HD), lambda i: (0, 0)),
            pl.BlockSpec((1, HD), lambda i: (0, 0)),
        ],
        out_specs=pl.BlockSpec((RB, GW), lambda i: (i, 0)),
        out_shape=jax.ShapeDtypeStruct((N, GW), jnp.float32),
    )(x, node_time, nn_W1, r2(nn_b1), nn_W2, r2(nn_b2), g_W1x, g_W1t)

    zeros = jnp.zeros((NPAD, HD), jnp.float32)
    eblk = EH // EB
    ea8 = edge_attr.reshape(E // 8, 8 * ED)
    eye8 = jnp.eye(8, dtype=jnp.float32)
    ew1_bd = jnp.einsum("ij,kl->ikjl", eye8, en_W1).reshape(8 * ED, 8 * HD)
    gwe_bd = jnp.einsum("ij,kl->ikjl", eye8, g_W1e).reshape(8 * ED, 8 * HD)
    gathered_halves = [_sc_gather(table, col[h]) for h in range(NH)]
    partials = []
    for h in range(NH):
        gathered = gathered_halves[h]
        msg = pl.pallas_call(
            _edge_body,
            grid=(eblk,),
            in_specs=[
                pl.BlockSpec((EB // 8, 8 * ED), lambda i, h=h: (i + h * eblk, 0)),
                pl.BlockSpec((EB, GW), lambda i: (i, 0)),
                pl.BlockSpec((8 * ED, 8 * HD), lambda i: (0, 0)),
                pl.BlockSpec((1, HD), lambda i: (0, 0)),
                pl.BlockSpec((HD, HD), lambda i: (0, 0)),
                pl.BlockSpec((1, HD), lambda i: (0, 0)),
                pl.BlockSpec((8 * ED, 8 * HD), lambda i: (0, 0)),
                pl.BlockSpec((1, HD), lambda i: (0, 0)),
                pl.BlockSpec((HD, HD), lambda i: (0, 0)),
                pl.BlockSpec((1, HD), lambda i: (0, 0)),
                pl.BlockSpec((HD, HD), lambda i: (0, 0)),
                pl.BlockSpec((1, HD), lambda i: (0, 0)),
            ],
            out_specs=pl.BlockSpec((EB, HD), lambda i: (i, 0)),
            out_shape=jax.ShapeDtypeStruct((EH, HD), jnp.float32),
        )(ea8, gathered, ew1_bd, r2(en_b1), en_W2, r2(en_b2),
          gwe_bd, r2(g_b1), g_W2, r2(g_b2), msg_W, r2(msg_b))
        partials.append(_sc_scatter(msg, row[h], zeros).reshape(NC, NPAD, HD))

    nblk = N // RB
    out = pl.pallas_call(
        _final_body,
        grid=(nblk,),
        in_specs=[
            pl.BlockSpec((RB, ND), lambda i: (i, 0)),
            pl.BlockSpec((1, RB, HD), lambda i: (0, i, 0)),
            pl.BlockSpec((1, RB, HD), lambda i: (1, i, 0)),
            pl.BlockSpec((1, RB, HD), lambda i: (0, i, 0)),
            pl.BlockSpec((1, RB, HD), lambda i: (1, i, 0)),
            pl.BlockSpec((ND, HD), lambda i: (0, 0)),
            pl.BlockSpec((1, HD), lambda i: (0, 0)),
            pl.BlockSpec((1, HD), lambda i: (0, 0)),
            pl.BlockSpec((1, HD), lambda i: (0, 0)),
            pl.BlockSpec((HD, ND), lambda i: (0, 0)),
            pl.BlockSpec((1, ND), lambda i: (0, 0)),
        ],
        out_specs=pl.BlockSpec((RB, ND), lambda i: (i, 0)),
        out_shape=jax.ShapeDtypeStruct((N, ND), jnp.float32),
    )(x, partials[0], partials[0], partials[1], partials[1],
      cl_W, r2(cl_b), r2(ln_g), r2(ln_b), out_W, r2(out_b))

    return out


# speedup vs baseline: 1.1092x; 1.1092x over previous
import functools

import jax
import jax.numpy as jnp
from jax import lax
from jax.experimental import pallas as pl
from jax.experimental.pallas import tpu as pltpu
from jax.experimental.pallas import tpu_sc as plsc

N = 10000
E = 320000
ND = 128
ED = 16
HD = 128
GW = HD

NC = 2
NS = 16
NW = NC * NS
NH = 2
EH = E // NH
EC = EH // NW
CH = 40
KCH = EC // CH
NPAD = 10240
NPT = NPAD // NS

RB = 2000
EB = 3200

_mesh = plsc.VectorSubcoreMesh(core_axis_name="c", subcore_axis_name="s")


def _node_pre_body(x_ref, nt_ref, w1, b1, w2, b2, gwx, gwt, out_ref):
    x = x_ref[...]
    h1 = jnp.maximum(jnp.dot(x, w1[...], preferred_element_type=jnp.float32)
                     + b1[...], 0.0)
    h_node = jnp.dot(h1, w2[...], preferred_element_type=jnp.float32) + b2[...]
    gp = (jnp.dot(x, gwx[...], preferred_element_type=jnp.float32)
          + nt_ref[...] * gwt[...])
    hu = jax.lax.bitcast_convert_type(h_node, jnp.uint32)
    gu = jax.lax.bitcast_convert_type(gp, jnp.uint32)
    rnd = lambda u: (u + jnp.uint32(0x7FFF) + ((u >> 16) & jnp.uint32(1))) >> 16
    packed = rnd(hu) | (rnd(gu) << 16)
    out_ref[...] = jax.lax.bitcast_convert_type(packed, jnp.float32)


def _edge_body(ea_ref, g_ref, ew1, eb1, ew2, eb2, gwe, gb1, gw2, gb2, mw, mb,
               out_ref):
    dot = functools.partial(jnp.dot, preferred_element_type=jnp.float32)
    ea8 = ea_ref[...]
    gu = jax.lax.bitcast_convert_type(g_ref[...], jnp.uint32)
    hn = jax.lax.bitcast_convert_type(gu << 16, jnp.float32)
    gp_node = jax.lax.bitcast_convert_type(gu & jnp.uint32(0xFFFF0000),
                                           jnp.float32)
    c1 = dot(ea8, ew1[...]).reshape(EB, HD)
    c2 = dot(ea8, gwe[...]).reshape(EB, HD)
    h1 = jnp.maximum(c1 + eb1[...], 0.0)
    he = dot(h1, ew2[...]) + eb2[...]
    gh = jnp.maximum(c2 + gp_node + gb1[...], 0.0)
    gate = dot(gh, gw2[...]) + gb2[...]
    m = dot(he * hn, mw[...]) + mb[...]
    out_ref[...] = m * jax.nn.sigmoid(gate)


def _final_body(x_ref, p0_ref, p1_ref, p2_ref, p3_ref, clw, clb, lng, lnb,
                ow, ob, out_ref):
    x = x_ref[...]
    o = (jnp.dot(x, clw[...], preferred_element_type=jnp.float32) + clb[...]
         + (p0_ref[0] + p1_ref[0]) + (p2_ref[0] + p3_ref[0]))
    mu = jnp.mean(o, axis=1, keepdims=True)
    var = jnp.mean((o - mu) * (o - mu), axis=1, keepdims=True)
    o = (o - mu) / jnp.sqrt(var + 1e-5) * lng[...] + lnb[...]
    o = jnp.maximum(o, 0.0)
    out_ref[...] = jnp.dot(o, ow[...], preferred_element_type=jnp.float32) + ob[...]


G = 5
GS = 5
CHS = 40
KCHS = EC // CHS


@functools.partial(
    pl.kernel,
    mesh=_mesh,
    out_type=jax.ShapeDtypeStruct((EH, GW), jnp.float32),
    scratch_types=[
        pltpu.VMEM((KCH, CH), jnp.int32),
        pltpu.VMEM((G, CH, GW), jnp.float32),
        pltpu.SemaphoreType.DMA,
    ],
)
def _sc_gather(table_hbm, idx_hbm, out_hbm, idx_v, rows_v, sem):
    c = lax.axis_index("c")
    s = lax.axis_index("s")
    t = c * NS + s
    pltpu.sync_copy(idx_hbm.at[t], idx_v)

    def group(g, carry):
        base = g * G
        handles = [
            pltpu.async_copy(table_hbm.at[idx_v.at[base + b]], rows_v.at[b],
                             sem)
            for b in range(G)
        ]
        for b in range(G):
            handles[b].wait()
            pltpu.sync_copy(rows_v.at[b],
                            out_hbm.at[pl.ds(t * EC + (base + b) * CH, CH)])
        return carry

    lax.fori_loop(0, KCH // G, group, 0)


@functools.partial(
    pl.kernel,
    mesh=_mesh,
    out_type=jax.ShapeDtypeStruct((NC * NPAD, HD), jnp.float32),
    scratch_types=[
        pltpu.VMEM((GS, CHS), jnp.int32),
        pltpu.VMEM((GS, CHS, HD), jnp.float32),
        pltpu.VMEM_SHARED((NPAD, HD), jnp.float32),
        pltpu.SemaphoreType.DMA,
        pltpu.SemaphoreType.DMA,
    ],
)
def _sc_scatter(msg_hbm, row_hbm, zero_hbm, out_hbm, idx_v, msg_v, acc_sh,
                sem, isem):
    c = lax.axis_index("c")
    s = lax.axis_index("s")
    t = c * NS + s
    pltpu.sync_copy(zero_hbm.at[pl.ds(s * NPT, NPT)],
                    acc_sh.at[pl.ds(s * NPT, NPT)])
    plsc.subcore_barrier()

    def group(g, carry):
        base = g * GS
        ih = pltpu.async_copy(row_hbm.at[t, g], idx_v, isem)
        handles = [
            pltpu.async_copy(
                msg_hbm.at[pl.ds(t * EC + (base + b) * CHS, CHS)],
                msg_v.at[b], sem)
            for b in range(GS)
        ]
        ih.wait()
        for b in range(GS):
            handles[b].wait()
            pltpu.sync_copy(msg_v.at[b], acc_sh.at[idx_v.at[b]],
                            add=True)
        return carry

    lax.fori_loop(0, KCHS // GS, group, 0)
    plsc.subcore_barrier()
    pltpu.sync_copy(acc_sh.at[pl.ds(s * NPT, NPT)],
                    out_hbm.at[pl.ds(c * NPAD + s * NPT, NPT)])


def kernel(x, edge_index, edge_attr, node_time,
           nn_W1, nn_b1, nn_W2, nn_b2,
           en_W1, en_b1, en_W2, en_b2,
           msg_W, msg_b,
           g_W1, g_b1, g_W2, g_b2,
           cl_W, cl_b, ln_g, ln_b, out_W, out_b):
    row = edge_index[0].reshape(NH, NW, KCHS // GS, GS, CHS)
    col = edge_index[1].reshape(NH, NW, KCH, CH)

    g_W1e = g_W1[:ED]
    g_W1x = g_W1[ED:ED + ND]
    g_W1t = g_W1[ED + ND:]

    r2 = lambda v: v.reshape(1, -1)

    table = pl.pallas_call(
        _node_pre_body,
        grid=(N // RB,),
        in_specs=[
            pl.BlockSpec((RB, ND), lambda i: (i, 0)),
            pl.BlockSpec((RB, 1), lambda i: (i, 0)),
            pl.BlockSpec((ND, HD), lambda i: (0, 0)),
            pl.BlockSpec((1, HD), lambda i: (0, 0)),
            pl.BlockSpec((HD, HD), lambda i: (0, 0)),
            pl.BlockSpec((1, HD), lambda i: (0, 0)),
            pl.BlockSpec((ND, HD), lambda i: (0, 0)),
            pl.BlockSpec((1, HD), lambda i: (0, 0)),
        ],
        out_specs=pl.BlockSpec((RB, GW), lambda i: (i, 0)),
        out_shape=jax.ShapeDtypeStruct((N, GW), jnp.float32),
    )(x, node_time, nn_W1, r2(nn_b1), nn_W2, r2(nn_b2), g_W1x, g_W1t)

    zeros = jnp.zeros((NPAD, HD), jnp.float32)
    eblk = EH // EB
    ea8 = edge_attr.reshape(E // 8, 8 * ED)
    eye8 = jnp.eye(8, dtype=jnp.float32)
    ew1_bd = jnp.einsum("ij,kl->ikjl", eye8, en_W1).reshape(8 * ED, 8 * HD)
    gwe_bd = jnp.einsum("ij,kl->ikjl", eye8, g_W1e).reshape(8 * ED, 8 * HD)
    gathered_halves = [_sc_gather(table, col[h]) for h in range(NH)]
    partials = []
    for h in range(NH):
        gathered = gathered_halves[h]
        msg = pl.pallas_call(
            _edge_body,
            grid=(eblk,),
            in_specs=[
                pl.BlockSpec((EB // 8, 8 * ED), lambda i, h=h: (i + h * eblk, 0)),
                pl.BlockSpec((EB, GW), lambda i: (i, 0)),
                pl.BlockSpec((8 * ED, 8 * HD), lambda i: (0, 0)),
                pl.BlockSpec((1, HD), lambda i: (0, 0)),
                pl.BlockSpec((HD, HD), lambda i: (0, 0)),
                pl.BlockSpec((1, HD), lambda i: (0, 0)),
                pl.BlockSpec((8 * ED, 8 * HD), lambda i: (0, 0)),
                pl.BlockSpec((1, HD), lambda i: (0, 0)),
                pl.BlockSpec((HD, HD), lambda i: (0, 0)),
                pl.BlockSpec((1, HD), lambda i: (0, 0)),
                pl.BlockSpec((HD, HD), lambda i: (0, 0)),
                pl.BlockSpec((1, HD), lambda i: (0, 0)),
            ],
            out_specs=pl.BlockSpec((EB, HD), lambda i: (i, 0)),
            out_shape=jax.ShapeDtypeStruct((EH, HD), jnp.float32),
        )(ea8, gathered, ew1_bd, r2(en_b1), en_W2, r2(en_b2),
          gwe_bd, r2(g_b1), g_W2, r2(g_b2), msg_W, r2(msg_b))
        partials.append(_sc_scatter(msg, row[h], zeros).reshape(NC, NPAD, HD))

    nblk = N // RB
    out = pl.pallas_call(
        _final_body,
        grid=(nblk,),
        in_specs=[
            pl.BlockSpec((RB, ND), lambda i: (i, 0)),
            pl.BlockSpec((1, RB, HD), lambda i: (0, i, 0)),
            pl.BlockSpec((1, RB, HD), lambda i: (1, i, 0)),
            pl.BlockSpec((1, RB, HD), lambda i: (0, i, 0)),
            pl.BlockSpec((1, RB, HD), lambda i: (1, i, 0)),
            pl.BlockSpec((ND, HD), lambda i: (0, 0)),
            pl.BlockSpec((1, HD), lambda i: (0, 0)),
            pl.BlockSpec((1, HD), lambda i: (0, 0)),
            pl.BlockSpec((1, HD), lambda i: (0, 0)),
            pl.BlockSpec((HD, ND), lambda i: (0, 0)),
            pl.BlockSpec((1, ND), lambda i: (0, 0)),
        ],
        out_specs=pl.BlockSpec((RB, ND), lambda i: (i, 0)),
        out_shape=jax.ShapeDtypeStruct((N, ND), jnp.float32),
    )(x, partials[0], partials[0], partials[1], partials[1],
      cl_W, r2(cl_b), r2(ln_g), r2(ln_b), out_W, r2(out_b))

    return out


# speedup vs baseline: 1.1585x; 1.0444x over previous
import functools

import jax
import jax.numpy as jnp
from jax import lax
from jax.experimental import pallas as pl
from jax.experimental.pallas import tpu as pltpu
from jax.experimental.pallas import tpu_sc as plsc

N = 10000
E = 320000
ND = 128
ED = 16
HD = 128
GW = HD

NC = 2
NS = 16
NW = NC * NS
NH = 2
EH = E // NH
EC = EH // NW
CH = 40
KCH = EC // CH
NPAD = 10240
NPT = NPAD // NS

RB = 2000
EB = 6400

_mesh = plsc.VectorSubcoreMesh(core_axis_name="c", subcore_axis_name="s")


def _node_pre_body(x_ref, nt_ref, w1, b1, w2, b2, gwx, gwt, out_ref):
    x = x_ref[...]
    h1 = jnp.maximum(jnp.dot(x, w1[...], preferred_element_type=jnp.float32)
                     + b1[...], 0.0)
    h_node = jnp.dot(h1, w2[...], preferred_element_type=jnp.float32) + b2[...]
    gp = (jnp.dot(x, gwx[...], preferred_element_type=jnp.float32)
          + nt_ref[...] * gwt[...])
    hu = jax.lax.bitcast_convert_type(h_node, jnp.uint32)
    gu = jax.lax.bitcast_convert_type(gp, jnp.uint32)
    rnd = lambda u: (u + jnp.uint32(0x7FFF) + ((u >> 16) & jnp.uint32(1))) >> 16
    packed = rnd(hu) | (rnd(gu) << 16)
    out_ref[...] = jax.lax.bitcast_convert_type(packed, jnp.float32)


def _edge_body(ea_ref, g_ref, ew1, eb1, ew2, eb2, gwe, gb1, gw2, gb2, mw, mb,
               out_ref):
    dot = functools.partial(jnp.dot, preferred_element_type=jnp.float32)
    ea8 = ea_ref[...]
    gu = jax.lax.bitcast_convert_type(g_ref[...], jnp.uint32)
    hn = jax.lax.bitcast_convert_type(gu << 16, jnp.float32)
    gp_node = jax.lax.bitcast_convert_type(gu & jnp.uint32(0xFFFF0000),
                                           jnp.float32)
    c1 = dot(ea8, ew1[...]).reshape(EB, HD)
    c2 = dot(ea8, gwe[...]).reshape(EB, HD)
    h1 = jnp.maximum(c1 + eb1[...], 0.0)
    he = dot(h1, ew2[...]) + eb2[...]
    gh = jnp.maximum(c2 + gp_node + gb1[...], 0.0)
    gate = dot(gh, gw2[...]) + gb2[...]
    m = dot(he * hn, mw[...]) + mb[...]
    out_ref[...] = m * jax.nn.sigmoid(gate)


def _final_body(x_ref, p0_ref, p1_ref, p2_ref, p3_ref, clw, clb, lng, lnb,
                ow, ob, out_ref):
    x = x_ref[...]
    o = (jnp.dot(x, clw[...], preferred_element_type=jnp.float32) + clb[...]
         + (p0_ref[0] + p1_ref[0]) + (p2_ref[0] + p3_ref[0]))
    mu = jnp.mean(o, axis=1, keepdims=True)
    var = jnp.mean((o - mu) * (o - mu), axis=1, keepdims=True)
    o = (o - mu) / jnp.sqrt(var + 1e-5) * lng[...] + lnb[...]
    o = jnp.maximum(o, 0.0)
    out_ref[...] = jnp.dot(o, ow[...], preferred_element_type=jnp.float32) + ob[...]


G = 5
GS = 5
CHS = 40
KCHS = EC // CHS


@functools.partial(
    pl.kernel,
    mesh=_mesh,
    out_type=jax.ShapeDtypeStruct((EH, GW), jnp.float32),
    scratch_types=[
        pltpu.VMEM((KCH, CH), jnp.int32),
        pltpu.VMEM((G, CH, GW), jnp.float32),
        pltpu.SemaphoreType.DMA,
    ],
)
def _sc_gather(table_hbm, idx_hbm, out_hbm, idx_v, rows_v, sem):
    c = lax.axis_index("c")
    s = lax.axis_index("s")
    t = c * NS + s
    pltpu.sync_copy(idx_hbm.at[t], idx_v)

    def group(g, carry):
        base = g * G
        handles = [
            pltpu.async_copy(table_hbm.at[idx_v.at[base + b]], rows_v.at[b],
                             sem)
            for b in range(G)
        ]
        for b in range(G):
            handles[b].wait()
            pltpu.sync_copy(rows_v.at[b],
                            out_hbm.at[pl.ds(t * EC + (base + b) * CH, CH)])
        return carry

    lax.fori_loop(0, KCH // G, group, 0)


@functools.partial(
    pl.kernel,
    mesh=_mesh,
    out_type=jax.ShapeDtypeStruct((NC * NPAD, HD), jnp.float32),
    scratch_types=[
        pltpu.VMEM((GS, CHS), jnp.int32),
        pltpu.VMEM((GS, CHS, HD), jnp.float32),
        pltpu.VMEM_SHARED((NPAD, HD), jnp.float32),
        pltpu.SemaphoreType.DMA,
        pltpu.SemaphoreType.DMA,
    ],
)
def _sc_scatter(msg_hbm, row_hbm, zero_hbm, out_hbm, idx_v, msg_v, acc_sh,
                sem, isem):
    c = lax.axis_index("c")
    s = lax.axis_index("s")
    t = c * NS + s
    pltpu.sync_copy(zero_hbm.at[pl.ds(s * NPT, NPT)],
                    acc_sh.at[pl.ds(s * NPT, NPT)])
    plsc.subcore_barrier()

    def group(g, carry):
        base = g * GS
        ih = pltpu.async_copy(row_hbm.at[t, g], idx_v, isem)
        handles = [
            pltpu.async_copy(
                msg_hbm.at[pl.ds(t * EC + (base + b) * CHS, CHS)],
                msg_v.at[b], sem)
            for b in range(GS)
        ]
        ih.wait()
        for b in range(GS):
            handles[b].wait()
            pltpu.sync_copy(msg_v.at[b], acc_sh.at[idx_v.at[b]],
                            add=True)
        return carry

    lax.fori_loop(0, KCHS // GS, group, 0)
    plsc.subcore_barrier()
    pltpu.sync_copy(acc_sh.at[pl.ds(s * NPT, NPT)],
                    out_hbm.at[pl.ds(c * NPAD + s * NPT, NPT)])


def kernel(x, edge_index, edge_attr, node_time,
           nn_W1, nn_b1, nn_W2, nn_b2,
           en_W1, en_b1, en_W2, en_b2,
           msg_W, msg_b,
           g_W1, g_b1, g_W2, g_b2,
           cl_W, cl_b, ln_g, ln_b, out_W, out_b):
    row = edge_index[0].reshape(NH, NW, KCHS // GS, GS, CHS)
    col = edge_index[1].reshape(NH, NW, KCH, CH)

    g_W1e = g_W1[:ED]
    g_W1x = g_W1[ED:ED + ND]
    g_W1t = g_W1[ED + ND:]

    r2 = lambda v: v.reshape(1, -1)

    table = pl.pallas_call(
        _node_pre_body,
        grid=(N // RB,),
        in_specs=[
            pl.BlockSpec((RB, ND), lambda i: (i, 0)),
            pl.BlockSpec((RB, 1), lambda i: (i, 0)),
            pl.BlockSpec((ND, HD), lambda i: (0, 0)),
            pl.BlockSpec((1, HD), lambda i: (0, 0)),
            pl.BlockSpec((HD, HD), lambda i: (0, 0)),
            pl.BlockSpec((1, HD), lambda i: (0, 0)),
            pl.BlockSpec((ND, HD), lambda i: (0, 0)),
            pl.BlockSpec((1, HD), lambda i: (0, 0)),
        ],
        out_specs=pl.BlockSpec((RB, GW), lambda i: (i, 0)),
        out_shape=jax.ShapeDtypeStruct((N, GW), jnp.float32),
    )(x, node_time, nn_W1, r2(nn_b1), nn_W2, r2(nn_b2), g_W1x, g_W1t)

    zeros = jnp.zeros((NPAD, HD), jnp.float32)
    eblk = EH // EB
    ea8 = edge_attr.reshape(E // 8, 8 * ED)
    eye8 = jnp.eye(8, dtype=jnp.float32)
    ew1_bd = jnp.einsum("ij,kl->ikjl", eye8, en_W1).reshape(8 * ED, 8 * HD)
    gwe_bd = jnp.einsum("ij,kl->ikjl", eye8, g_W1e).reshape(8 * ED, 8 * HD)
    gathered_halves = [_sc_gather(table, col[h]) for h in range(NH)]
    partials = []
    for h in range(NH):
        gathered = gathered_halves[h]
        msg = pl.pallas_call(
            _edge_body,
            grid=(eblk,),
            in_specs=[
                pl.BlockSpec((EB // 8, 8 * ED), lambda i, h=h: (i + h * eblk, 0)),
                pl.BlockSpec((EB, GW), lambda i: (i, 0)),
                pl.BlockSpec((8 * ED, 8 * HD), lambda i: (0, 0)),
                pl.BlockSpec((1, HD), lambda i: (0, 0)),
                pl.BlockSpec((HD, HD), lambda i: (0, 0)),
                pl.BlockSpec((1, HD), lambda i: (0, 0)),
                pl.BlockSpec((8 * ED, 8 * HD), lambda i: (0, 0)),
                pl.BlockSpec((1, HD), lambda i: (0, 0)),
                pl.BlockSpec((HD, HD), lambda i: (0, 0)),
                pl.BlockSpec((1, HD), lambda i: (0, 0)),
                pl.BlockSpec((HD, HD), lambda i: (0, 0)),
                pl.BlockSpec((1, HD), lambda i: (0, 0)),
            ],
            out_specs=pl.BlockSpec((EB, HD), lambda i: (i, 0)),
            out_shape=jax.ShapeDtypeStruct((EH, HD), jnp.float32),
        )(ea8, gathered, ew1_bd, r2(en_b1), en_W2, r2(en_b2),
          gwe_bd, r2(g_b1), g_W2, r2(g_b2), msg_W, r2(msg_b))
        partials.append(_sc_scatter(msg, row[h], zeros).reshape(NC, NPAD, HD))

    nblk = N // RB
    out = pl.pallas_call(
        _final_body,
        grid=(nblk,),
        in_specs=[
            pl.BlockSpec((RB, ND), lambda i: (i, 0)),
            pl.BlockSpec((1, RB, HD), lambda i: (0, i, 0)),
            pl.BlockSpec((1, RB, HD), lambda i: (1, i, 0)),
            pl.BlockSpec((1, RB, HD), lambda i: (0, i, 0)),
            pl.BlockSpec((1, RB, HD), lambda i: (1, i, 0)),
            pl.BlockSpec((ND, HD), lambda i: (0, 0)),
            pl.BlockSpec((1, HD), lambda i: (0, 0)),
            pl.BlockSpec((1, HD), lambda i: (0, 0)),
            pl.BlockSpec((1, HD), lambda i: (0, 0)),
            pl.BlockSpec((HD, ND), lambda i: (0, 0)),
            pl.BlockSpec((1, ND), lambda i: (0, 0)),
        ],
        out_specs=pl.BlockSpec((RB, ND), lambda i: (i, 0)),
        out_shape=jax.ShapeDtypeStruct((N, ND), jnp.float32),
    )(x, partials[0], partials[0], partials[1], partials[1],
      cl_W, r2(cl_b), r2(ln_g), r2(ln_b), out_W, r2(out_b))

    return out


# speedup vs baseline: 1.1640x; 1.0048x over previous
import functools

import jax
import jax.numpy as jnp
from jax import lax
from jax.experimental import pallas as pl
from jax.experimental.pallas import tpu as pltpu
from jax.experimental.pallas import tpu_sc as plsc

N = 10000
E = 320000
ND = 128
ED = 16
HD = 128
GW = HD

NC = 2
NS = 16
NW = NC * NS
NH = 2
EH = E // NH
EC = EH // NW
CH = 40
KCH = EC // CH
NPAD = 10240
NPT = NPAD // NS

RB = 2000
EB = 8000

_mesh = plsc.VectorSubcoreMesh(core_axis_name="c", subcore_axis_name="s")


def _node_pre_body(x_ref, nt_ref, w1, b1, w2, b2, gwx, gwt, out_ref):
    x = x_ref[...]
    h1 = jnp.maximum(jnp.dot(x, w1[...], preferred_element_type=jnp.float32)
                     + b1[...], 0.0)
    h_node = jnp.dot(h1, w2[...], preferred_element_type=jnp.float32) + b2[...]
    gp = (jnp.dot(x, gwx[...], preferred_element_type=jnp.float32)
          + nt_ref[...] * gwt[...])
    hu = jax.lax.bitcast_convert_type(h_node, jnp.uint32)
    gu = jax.lax.bitcast_convert_type(gp, jnp.uint32)
    rnd = lambda u: (u + jnp.uint32(0x7FFF) + ((u >> 16) & jnp.uint32(1))) >> 16
    packed = rnd(hu) | (rnd(gu) << 16)
    out_ref[...] = jax.lax.bitcast_convert_type(packed, jnp.float32)


def _edge_body(ea_ref, g_ref, ew1, eb1, ew2, eb2, gwe, gb1, gw2, gb2, mw, mb,
               out_ref):
    dot = functools.partial(jnp.dot, preferred_element_type=jnp.float32)
    ea8 = ea_ref[...]
    gu = jax.lax.bitcast_convert_type(g_ref[...], jnp.uint32)
    hn = jax.lax.bitcast_convert_type(gu << 16, jnp.float32)
    gp_node = jax.lax.bitcast_convert_type(gu & jnp.uint32(0xFFFF0000),
                                           jnp.float32)
    c1 = dot(ea8, ew1[...]).reshape(EB, HD)
    c2 = dot(ea8, gwe[...]).reshape(EB, HD)
    h1 = jnp.maximum(c1 + eb1[...], 0.0)
    he = dot(h1, ew2[...]) + eb2[...]
    gh = jnp.maximum(c2 + gp_node + gb1[...], 0.0)
    gate = dot(gh, gw2[...]) + gb2[...]
    m = dot(he * hn, mw[...]) + mb[...]
    out_ref[...] = m * jax.nn.sigmoid(gate)


def _final_body(x_ref, p0_ref, p1_ref, p2_ref, p3_ref, clw, clb, lng, lnb,
                ow, ob, out_ref):
    x = x_ref[...]
    o = (jnp.dot(x, clw[...], preferred_element_type=jnp.float32) + clb[...]
         + (p0_ref[0] + p1_ref[0]) + (p2_ref[0] + p3_ref[0]))
    mu = jnp.mean(o, axis=1, keepdims=True)
    var = jnp.mean((o - mu) * (o - mu), axis=1, keepdims=True)
    o = (o - mu) / jnp.sqrt(var + 1e-5) * lng[...] + lnb[...]
    o = jnp.maximum(o, 0.0)
    out_ref[...] = jnp.dot(o, ow[...], preferred_element_type=jnp.float32) + ob[...]


G = 5
GS = 5
CHS = 40
KCHS = EC // CHS


@functools.partial(
    pl.kernel,
    mesh=_mesh,
    out_type=jax.ShapeDtypeStruct((EH, GW), jnp.float32),
    scratch_types=[
        pltpu.VMEM((KCH, CH), jnp.int32),
        pltpu.VMEM((G, CH, GW), jnp.float32),
        pltpu.SemaphoreType.DMA,
    ],
)
def _sc_gather(table_hbm, idx_hbm, out_hbm, idx_v, rows_v, sem):
    c = lax.axis_index("c")
    s = lax.axis_index("s")
    t = c * NS + s
    pltpu.sync_copy(idx_hbm.at[t], idx_v)

    def group(g, carry):
        base = g * G
        handles = [
            pltpu.async_copy(table_hbm.at[idx_v.at[base + b]], rows_v.at[b],
                             sem)
            for b in range(G)
        ]
        for b in range(G):
            handles[b].wait()
            pltpu.sync_copy(rows_v.at[b],
                            out_hbm.at[pl.ds(t * EC + (base + b) * CH, CH)])
        return carry

    lax.fori_loop(0, KCH // G, group, 0)


@functools.partial(
    pl.kernel,
    mesh=_mesh,
    out_type=jax.ShapeDtypeStruct((NC * NPAD, HD), jnp.float32),
    scratch_types=[
        pltpu.VMEM((GS, CHS), jnp.int32),
        pltpu.VMEM((GS, CHS, HD), jnp.float32),
        pltpu.VMEM_SHARED((NPAD, HD), jnp.float32),
        pltpu.SemaphoreType.DMA,
        pltpu.SemaphoreType.DMA,
    ],
)
def _sc_scatter(msg_hbm, row_hbm, zero_hbm, out_hbm, idx_v, msg_v, acc_sh,
                sem, isem):
    c = lax.axis_index("c")
    s = lax.axis_index("s")
    t = c * NS + s
    pltpu.sync_copy(zero_hbm.at[pl.ds(s * NPT, NPT)],
                    acc_sh.at[pl.ds(s * NPT, NPT)])
    plsc.subcore_barrier()

    def group(g, carry):
        base = g * GS
        ih = pltpu.async_copy(row_hbm.at[t, g], idx_v, isem)
        handles = [
            pltpu.async_copy(
                msg_hbm.at[pl.ds(t * EC + (base + b) * CHS, CHS)],
                msg_v.at[b], sem)
            for b in range(GS)
        ]
        ih.wait()
        for b in range(GS):
            handles[b].wait()
            pltpu.sync_copy(msg_v.at[b], acc_sh.at[idx_v.at[b]],
                            add=True)
        return carry

    lax.fori_loop(0, KCHS // GS, group, 0)
    plsc.subcore_barrier()
    pltpu.sync_copy(acc_sh.at[pl.ds(s * NPT, NPT)],
                    out_hbm.at[pl.ds(c * NPAD + s * NPT, NPT)])


def kernel(x, edge_index, edge_attr, node_time,
           nn_W1, nn_b1, nn_W2, nn_b2,
           en_W1, en_b1, en_W2, en_b2,
           msg_W, msg_b,
           g_W1, g_b1, g_W2, g_b2,
           cl_W, cl_b, ln_g, ln_b, out_W, out_b):
    row = edge_index[0].reshape(NH, NW, KCHS // GS, GS, CHS)
    col = edge_index[1].reshape(NH, NW, KCH, CH)

    g_W1e = g_W1[:ED]
    g_W1x = g_W1[ED:ED + ND]
    g_W1t = g_W1[ED + ND:]

    r2 = lambda v: v.reshape(1, -1)

    table = pl.pallas_call(
        _node_pre_body,
        grid=(N // RB,),
        in_specs=[
            pl.BlockSpec((RB, ND), lambda i: (i, 0)),
            pl.BlockSpec((RB, 1), lambda i: (i, 0)),
            pl.BlockSpec((ND, HD), lambda i: (0, 0)),
            pl.BlockSpec((1, HD), lambda i: (0, 0)),
            pl.BlockSpec((HD, HD), lambda i: (0, 0)),
            pl.BlockSpec((1, HD), lambda i: (0, 0)),
            pl.BlockSpec((ND, HD), lambda i: (0, 0)),
            pl.BlockSpec((1, HD), lambda i: (0, 0)),
        ],
        out_specs=pl.BlockSpec((RB, GW), lambda i: (i, 0)),
        out_shape=jax.ShapeDtypeStruct((N, GW), jnp.float32),
    )(x, node_time, nn_W1, r2(nn_b1), nn_W2, r2(nn_b2), g_W1x, g_W1t)

    zeros = jnp.zeros((NPAD, HD), jnp.float32)
    eblk = EH // EB
    ea8 = edge_attr.reshape(E // 8, 8 * ED)
    eye8 = jnp.eye(8, dtype=jnp.float32)
    ew1_bd = jnp.einsum("ij,kl->ikjl", eye8, en_W1).reshape(8 * ED, 8 * HD)
    gwe_bd = jnp.einsum("ij,kl->ikjl", eye8, g_W1e).reshape(8 * ED, 8 * HD)
    gathered_halves = [_sc_gather(table, col[h]) for h in range(NH)]
    partials = []
    for h in range(NH):
        gathered = gathered_halves[h]
        msg = pl.pallas_call(
            _edge_body,
            grid=(eblk,),
            in_specs=[
                pl.BlockSpec((EB // 8, 8 * ED), lambda i, h=h: (i + h * eblk, 0)),
                pl.BlockSpec((EB, GW), lambda i: (i, 0)),
                pl.BlockSpec((8 * ED, 8 * HD), lambda i: (0, 0)),
                pl.BlockSpec((1, HD), lambda i: (0, 0)),
                pl.BlockSpec((HD, HD), lambda i: (0, 0)),
                pl.BlockSpec((1, HD), lambda i: (0, 0)),
                pl.BlockSpec((8 * ED, 8 * HD), lambda i: (0, 0)),
                pl.BlockSpec((1, HD), lambda i: (0, 0)),
                pl.BlockSpec((HD, HD), lambda i: (0, 0)),
                pl.BlockSpec((1, HD), lambda i: (0, 0)),
                pl.BlockSpec((HD, HD), lambda i: (0, 0)),
                pl.BlockSpec((1, HD), lambda i: (0, 0)),
            ],
            out_specs=pl.BlockSpec((EB, HD), lambda i: (i, 0)),
            out_shape=jax.ShapeDtypeStruct((EH, HD), jnp.float32),
        )(ea8, gathered, ew1_bd, r2(en_b1), en_W2, r2(en_b2),
          gwe_bd, r2(g_b1), g_W2, r2(g_b2), msg_W, r2(msg_b))
        partials.append(_sc_scatter(msg, row[h], zeros).reshape(NC, NPAD, HD))

    nblk = N // RB
    out = pl.pallas_call(
        _final_body,
        grid=(nblk,),
        in_specs=[
            pl.BlockSpec((RB, ND), lambda i: (i, 0)),
            pl.BlockSpec((1, RB, HD), lambda i: (0, i, 0)),
            pl.BlockSpec((1, RB, HD), lambda i: (1, i, 0)),
            pl.BlockSpec((1, RB, HD), lambda i: (0, i, 0)),
            pl.BlockSpec((1, RB, HD), lambda i: (1, i, 0)),
            pl.BlockSpec((ND, HD), lambda i: (0, 0)),
            pl.BlockSpec((1, HD), lambda i: (0, 0)),
            pl.BlockSpec((1, HD), lambda i: (0, 0)),
            pl.BlockSpec((1, HD), lambda i: (0, 0)),
            pl.BlockSpec((HD, ND), lambda i: (0, 0)),
            pl.BlockSpec((1, ND), lambda i: (0, 0)),
        ],
        out_specs=pl.BlockSpec((RB, ND), lambda i: (i, 0)),
        out_shape=jax.ShapeDtypeStruct((N, ND), jnp.float32),
    )(x, partials[0], partials[0], partials[1], partials[1],
      cl_W, r2(cl_b), r2(ln_g), r2(ln_b), out_W, r2(out_b))

    return out


# speedup vs baseline: 1.1652x; 1.0010x over previous
import functools

import jax
import jax.numpy as jnp
from jax import lax
from jax.experimental import pallas as pl
from jax.experimental.pallas import tpu as pltpu
from jax.experimental.pallas import tpu_sc as plsc

N = 10000
E = 320000
ND = 128
ED = 16
HD = 128
GW = HD

NC = 2
NS = 16
NW = NC * NS
NH = 2
EH = E // NH
EC = EH // NW
CH = 40
KCH = EC // CH
NPAD = 10240
NPT = NPAD // NS

RB = 2000
EB = 8000

_mesh = plsc.VectorSubcoreMesh(core_axis_name="c", subcore_axis_name="s")


def _node_pre_body(x_ref, nt_ref, w1, b1, w2, b2, gwx, gwt, out_ref):
    x = x_ref[...]
    h1 = jnp.maximum(jnp.dot(x, w1[...], preferred_element_type=jnp.float32)
                     + b1[...], 0.0)
    h_node = jnp.dot(h1, w2[...], preferred_element_type=jnp.float32) + b2[...]
    gp = (jnp.dot(x, gwx[...], preferred_element_type=jnp.float32)
          + nt_ref[...] * gwt[...])
    hu = jax.lax.bitcast_convert_type(h_node, jnp.uint32)
    gu = jax.lax.bitcast_convert_type(gp, jnp.uint32)
    rnd = lambda u: (u + jnp.uint32(0x7FFF) + ((u >> 16) & jnp.uint32(1))) >> 16
    packed = rnd(hu) | (rnd(gu) << 16)
    out_ref[...] = jax.lax.bitcast_convert_type(packed, jnp.float32)


def _edge_body(ea_ref, g_ref, ew1, eb1, ew2, eb2, gwe, gb1, gw2, gb2, mw, mb,
               out_ref):
    dot = functools.partial(jnp.dot, preferred_element_type=jnp.float32)
    ea8 = ea_ref[...]
    gu = jax.lax.bitcast_convert_type(g_ref[...], jnp.uint32)
    hn = jax.lax.bitcast_convert_type(gu << 16, jnp.float32)
    gp_node = jax.lax.bitcast_convert_type(gu & jnp.uint32(0xFFFF0000),
                                           jnp.float32)
    c1 = dot(ea8, ew1[...]).reshape(EB, HD)
    c2 = dot(ea8, gwe[...]).reshape(EB, HD)
    h1 = jnp.maximum(c1 + eb1[...], 0.0)
    he = dot(h1, ew2[...]) + eb2[...]
    gh = jnp.maximum(c2 + gp_node + gb1[...], 0.0)
    gate = dot(gh, gw2[...]) + gb2[...]
    m = dot(he * hn, mw[...]) + mb[...]
    out_ref[...] = m * jax.nn.sigmoid(gate)


def _final_body(x_ref, p0_ref, p1_ref, p2_ref, p3_ref, clw, clb, lng, lnb,
                ow, ob, out_ref):
    x = x_ref[...]
    o = (jnp.dot(x, clw[...], preferred_element_type=jnp.float32) + clb[...]
         + (p0_ref[0] + p1_ref[0]) + (p2_ref[0] + p3_ref[0]))
    mu = jnp.mean(o, axis=1, keepdims=True)
    var = jnp.mean((o - mu) * (o - mu), axis=1, keepdims=True)
    o = (o - mu) / jnp.sqrt(var + 1e-5) * lng[...] + lnb[...]
    o = jnp.maximum(o, 0.0)
    out_ref[...] = jnp.dot(o, ow[...], preferred_element_type=jnp.float32) + ob[...]


G = 5
GS = 5
CHS = 40
KCHS = EC // CHS


@functools.partial(
    pl.kernel,
    mesh=_mesh,
    out_type=jax.ShapeDtypeStruct((EH, GW), jnp.float32),
    scratch_types=[
        pltpu.VMEM((KCH, CH), jnp.int32),
        pltpu.VMEM((G, CH, GW), jnp.float32),
        pltpu.SemaphoreType.DMA,
    ],
)
def _sc_gather(table_hbm, idx_hbm, out_hbm, idx_v, rows_v, sem):
    c = lax.axis_index("c")
    s = lax.axis_index("s")
    t = c * NS + s
    pltpu.sync_copy(idx_hbm.at[t], idx_v)

    def group(g, carry):
        base = g * G
        handles = [
            pltpu.async_copy(table_hbm.at[idx_v.at[base + b]], rows_v.at[b],
                             sem)
            for b in range(G)
        ]
        for b in range(G):
            handles[b].wait()
            pltpu.sync_copy(rows_v.at[b],
                            out_hbm.at[pl.ds(t * EC + (base + b) * CH, CH)])
        return carry

    lax.fori_loop(0, KCH // G, group, 0)


@functools.partial(
    pl.kernel,
    mesh=_mesh,
    out_type=jax.ShapeDtypeStruct((NC * NPAD, HD), jnp.float32),
    scratch_types=[
        pltpu.VMEM((GS, CHS), jnp.int32),
        pltpu.VMEM((GS, CHS, HD), jnp.float32),
        pltpu.VMEM_SHARED((NPAD, HD), jnp.float32),
        pltpu.SemaphoreType.DMA,
        pltpu.SemaphoreType.DMA,
    ],
)
def _sc_scatter(msg_hbm, row_hbm, zero_hbm, out_hbm, idx_v, msg_v, acc_sh,
                sem, isem):
    c = lax.axis_index("c")
    s = lax.axis_index("s")
    t = c * NS + s
    pltpu.sync_copy(zero_hbm.at[pl.ds(s * NPT, NPT)],
                    acc_sh.at[pl.ds(s * NPT, NPT)])
    plsc.subcore_barrier()

    def group(g, carry):
        base = g * GS
        ih = pltpu.async_copy(row_hbm.at[t, g], idx_v, isem)
        handles = [
            pltpu.async_copy(
                msg_hbm.at[pl.ds(t * EC + (base + b) * CHS, CHS)],
                msg_v.at[b], sem)
            for b in range(GS)
        ]
        ih.wait()
        for b in range(GS):
            handles[b].wait()
            pltpu.sync_copy(msg_v.at[b], acc_sh.at[idx_v.at[b]],
                            add=True)
        return carry

    lax.fori_loop(0, KCHS // GS, group, 0)
    plsc.subcore_barrier()
    pltpu.sync_copy(acc_sh.at[pl.ds(s * NPT, NPT)],
                    out_hbm.at[pl.ds(c * NPAD + s * NPT, NPT)])


def kernel(x, edge_index, edge_attr, node_time,
           nn_W1, nn_b1, nn_W2, nn_b2,
           en_W1, en_b1, en_W2, en_b2,
           msg_W, msg_b,
           g_W1, g_b1, g_W2, g_b2,
           cl_W, cl_b, ln_g, ln_b, out_W, out_b):
    ea8 = edge_attr.reshape(E // 8, 8 * ED)

    row = edge_index[0].reshape(NH, NW, KCHS // GS, GS, CHS)
    col = edge_index[1].reshape(NH, NW, KCH, CH)

    g_W1e = g_W1[:ED]
    g_W1x = g_W1[ED:ED + ND]
    g_W1t = g_W1[ED + ND:]

    r2 = lambda v: v.reshape(1, -1)

    table = pl.pallas_call(
        _node_pre_body,
        grid=(N // RB,),
        in_specs=[
            pl.BlockSpec((RB, ND), lambda i: (i, 0)),
            pl.BlockSpec((RB, 1), lambda i: (i, 0)),
            pl.BlockSpec((ND, HD), lambda i: (0, 0)),
            pl.BlockSpec((1, HD), lambda i: (0, 0)),
            pl.BlockSpec((HD, HD), lambda i: (0, 0)),
            pl.BlockSpec((1, HD), lambda i: (0, 0)),
            pl.BlockSpec((ND, HD), lambda i: (0, 0)),
            pl.BlockSpec((1, HD), lambda i: (0, 0)),
        ],
        out_specs=pl.BlockSpec((RB, GW), lambda i: (i, 0)),
        out_shape=jax.ShapeDtypeStruct((N, GW), jnp.float32),
    )(x, node_time, nn_W1, r2(nn_b1), nn_W2, r2(nn_b2), g_W1x, g_W1t)

    zeros = jnp.zeros((NPAD, HD), jnp.float32)
    eblk = EH // EB
    eye8 = jnp.eye(8, dtype=jnp.float32)
    ew1_bd = jnp.einsum("ij,kl->ikjl", eye8, en_W1).reshape(8 * ED, 8 * HD)
    gwe_bd = jnp.einsum("ij,kl->ikjl", eye8, g_W1e).reshape(8 * ED, 8 * HD)
    gathered_halves = [_sc_gather(table, col[h]) for h in range(NH)]
    partials = []
    for h in range(NH):
        gathered = gathered_halves[h]
        msg = pl.pallas_call(
            _edge_body,
            grid=(eblk,),
            in_specs=[
                pl.BlockSpec((EB // 8, 8 * ED), lambda i, h=h: (i + h * eblk, 0)),
                pl.BlockSpec((EB, GW), lambda i: (i, 0)),
                pl.BlockSpec((8 * ED, 8 * HD), lambda i: (0, 0)),
                pl.BlockSpec((1, HD), lambda i: (0, 0)),
                pl.BlockSpec((HD, HD), lambda i: (0, 0)),
                pl.BlockSpec((1, HD), lambda i: (0, 0)),
                pl.BlockSpec((8 * ED, 8 * HD), lambda i: (0, 0)),
                pl.BlockSpec((1, HD), lambda i: (0, 0)),
                pl.BlockSpec((HD, HD), lambda i: (0, 0)),
                pl.BlockSpec((1, HD), lambda i: (0, 0)),
                pl.BlockSpec((HD, HD), lambda i: (0, 0)),
                pl.BlockSpec((1, HD), lambda i: (0, 0)),
            ],
            out_specs=pl.BlockSpec((EB, HD), lambda i: (i, 0)),
            out_shape=jax.ShapeDtypeStruct((EH, HD), jnp.float32),
        )(ea8, gathered, ew1_bd, r2(en_b1), en_W2, r2(en_b2),
          gwe_bd, r2(g_b1), g_W2, r2(g_b2), msg_W, r2(msg_b))
        partials.append(_sc_scatter(msg, row[h], zeros).reshape(NC, NPAD, HD))

    nblk = N // RB
    out = pl.pallas_call(
        _final_body,
        grid=(nblk,),
        in_specs=[
            pl.BlockSpec((RB, ND), lambda i: (i, 0)),
            pl.BlockSpec((1, RB, HD), lambda i: (0, i, 0)),
            pl.BlockSpec((1, RB, HD), lambda i: (1, i, 0)),
            pl.BlockSpec((1, RB, HD), lambda i: (0, i, 0)),
            pl.BlockSpec((1, RB, HD), lambda i: (1, i, 0)),
            pl.BlockSpec((ND, HD), lambda i: (0, 0)),
            pl.BlockSpec((1, HD), lambda i: (0, 0)),
            pl.BlockSpec((1, HD), lambda i: (0, 0)),
            pl.BlockSpec((1, HD), lambda i: (0, 0)),
            pl.BlockSpec((HD, ND), lambda i: (0, 0)),
            pl.BlockSpec((1, ND), lambda i: (0, 0)),
        ],
        out_specs=pl.BlockSpec((RB, ND), lambda i: (i, 0)),
        out_shape=jax.ShapeDtypeStruct((N, ND), jnp.float32),
    )(x, partials[0], partials[0], partials[1], partials[1],
      cl_W, r2(cl_b), r2(ln_g), r2(ln_b), out_W, r2(out_b))

    return out
